# Initial kernel scaffold; baseline (speedup 1.0000x reference)
import jax
import jax.numpy as jnp
from jax.experimental import pallas as pl


def kernel(x, token_table, pos_table):
    raise NotImplementedError("write your pallas kernel here")


# speedup vs baseline: 3.9530x; 3.9530x over previous
import functools

import jax
import jax.numpy as jnp
from jax import lax
from jax.experimental import pallas as pl
from jax.experimental.pallas import tpu as pltpu
from jax.experimental.pallas import tpu_sc as plsc


def _make_sc_kernel(B, S, V, D, NC=2, NS=16, L=16, interpret=False):
  NW = NC * NS
  SPW = B // NW
  H = S // 2
  mesh = plsc.VectorSubcoreMesh(core_axis_name="c", subcore_axis_name="s",
                                num_cores=NC, num_subcores=NS)

  @functools.partial(
      pl.kernel,
      out_type=jax.ShapeDtypeStruct((B, S, D), jnp.float32),
      mesh=mesh,
      scratch_types=[
          pltpu.VMEM((S, D), jnp.float32),
          pltpu.VMEM((2, H), jnp.int32),
          pltpu.VMEM((S, D), jnp.float32),
          pltpu.SemaphoreType.DMA,
      ],
      interpret=interpret,
  )
  def k(x_hbm, tok_hbm, pos_hbm, out_hbm, pos_v, idx_v, rows_v, sem):
    wid = lax.axis_index("s") * NC + lax.axis_index("c")
    pltpu.sync_copy(pos_hbm, pos_v)

    def seq_body(i, carry):
      seq = wid * SPW + i
      pltpu.sync_copy(x_hbm.at[seq], idx_v)
      cp0 = pltpu.async_copy(tok_hbm.at[idx_v.at[0]], rows_v.at[pl.ds(0, H)], sem)
      cp1 = pltpu.async_copy(tok_hbm.at[idx_v.at[1]], rows_v.at[pl.ds(H, H)], sem)
      cp0.wait()
      cp1.wait()

      def add_body(r, c):
        for l in range(D // L):
          sl = pl.ds(l * L, L)
          rows_v[r, sl] = rows_v[r, sl] + pos_v[r, sl]
        return c

      lax.fori_loop(0, S, add_body, 0)
      pltpu.sync_copy(rows_v, out_hbm.at[seq])
      return carry

    lax.fori_loop(0, SPW, seq_body, 0)

  return k


def kernel(x, token_table, pos_table):
  B, S = x.shape
  V, D = token_table.shape
  x32 = x.astype(jnp.int32).reshape(B, 2, S // 2)
  info = plsc.get_sparse_core_info()
  k = _make_sc_kernel(B, S, V, D, NC=info.num_cores, NS=info.num_subcores,
                      L=info.num_lanes)
  return k(x32, token_table, pos_table)


# speedup vs baseline: 6.3980x; 1.6185x over previous
import functools

import jax
import jax.numpy as jnp
from jax import lax
from jax.experimental import pallas as pl
from jax.experimental.pallas import tpu as pltpu
from jax.experimental.pallas import tpu_sc as plsc


def _make_sc_kernel(B, S, V, D, NC=2, NS=16, L=16, interpret=False):
  NW = NC * NS
  SPW = B // NW
  H = S // 2
  mesh = plsc.VectorSubcoreMesh(core_axis_name="c", subcore_axis_name="s",
                                num_cores=NC, num_subcores=NS)

  @functools.partial(
      pl.kernel,
      out_type=jax.ShapeDtypeStruct((B, S, D), jnp.float32),
      mesh=mesh,
      scratch_types=[
          pltpu.VMEM((S, D), jnp.float32),
          pltpu.VMEM((SPW, 2, H), jnp.int32),
          pltpu.VMEM((2, S, D), jnp.float32),
          pltpu.SemaphoreType.DMA((2,)),
          pltpu.SemaphoreType.DMA((2,)),
      ],
      interpret=interpret,
  )
  def k(x_hbm, tok_hbm, pos_hbm, out_hbm, pos_v, idx_v, rows_v, gsem, osem):
    wid = lax.axis_index("s") * NC + lax.axis_index("c")
    pltpu.sync_copy(x_hbm.at[wid], idx_v)
    pltpu.sync_copy(pos_hbm, pos_v)

    def start_gather(i, b):
      pltpu.async_copy(tok_hbm.at[idx_v.at[i, 0]], rows_v.at[b, pl.ds(0, H)],
                       gsem.at[b])
      pltpu.async_copy(tok_hbm.at[idx_v.at[i, 1]], rows_v.at[b, pl.ds(H, H)],
                       gsem.at[b])

    def wait_gather(b):
      pltpu.make_async_copy(tok_hbm.at[pl.ds(0, S)], rows_v.at[b],
                            gsem.at[b]).wait()

    def wait_writeback(b, seq):
      pltpu.make_async_copy(rows_v.at[b], out_hbm.at[seq], osem.at[b]).wait()

    start_gather(0, 0)

    def body(g2, carry):
      for b in range(2):
        i = g2 * 2 + b
        seq = wid * SPW + i
        nb = 1 - b
        if b == 0:
          @pl.when(g2 >= 1)
          def _():
            wait_writeback(nb, seq)
          start_gather(i + 1, nb)
        else:
          @pl.when(g2 < SPW // 2 - 1)
          def _():
            wait_writeback(nb, seq)
            start_gather(i + 1, nb)
        wait_gather(b)

        def add_body(r, c):
          for l in range(D // L):
            sl = pl.ds(l * L, L)
            rows_v[b, r, sl] = rows_v[b, r, sl] + pos_v[r, sl]
          return c

        lax.fori_loop(0, S, add_body, 0)
        pltpu.async_copy(rows_v.at[b], out_hbm.at[seq], osem.at[b])
      return carry

    lax.fori_loop(0, SPW // 2, body, 0)
    last = wid * SPW + SPW - 1
    wait_writeback(0, last)
    wait_writeback(1, last)

  return k


def kernel(x, token_table, pos_table):
  B, S = x.shape
  V, D = token_table.shape
  info = plsc.get_sparse_core_info()
  NC, NS, L = info.num_cores, info.num_subcores, info.num_lanes
  NW = NC * NS
  x32 = x.astype(jnp.int32).reshape(NW, B // NW, 2, S // 2)
  k = _make_sc_kernel(B, S, V, D, NC=NC, NS=NS, L=L)
  return k(x32, token_table, pos_table)
